# baseline (device time: 126611 ns/iter reference)
import functools

import jax
import jax.numpy as jnp
from jax import lax
from jax.experimental import pallas as pl
from jax.experimental.pallas import tpu as pltpu

N_DEV = 4
FP8 = jnp.float8_e4m3fn
NC = 4


def kernel(x, w_mat, scale_x, scale_w):
    m_per, k = x.shape
    n = w_mat.shape[1]
    n_per = n // N_DEV
    m_half = m_per // 2
    m_total = m_per * N_DEV
    cr = m_per // NC
    ccols = 256
    n_wchunks = n_per // ccols

    xq = x.astype(FP8)
    out_init = jnp.zeros((m_total, n_per), jnp.float32)

    def body(x_ref, w_ref, sx_ref, sw_ref, dummy_ref, out_ref,
             wstage, wq, buf_l, buf_r, buf_o, ostage,
             ssl, ssr, sfr, sfl, rl, rr, ro, wsems, osems):
        my = lax.axis_index("i")
        left = lax.rem(my + N_DEV - 1, N_DEV)
        right = lax.rem(my + 1, N_DEV)
        opp = lax.rem(my + 2, N_DEV)

        def w_chunk_copy(c, slot):
            return pltpu.make_async_copy(
                w_ref.at[:, pl.ds(my * n_per + c * ccols, ccols)],
                wstage.at[slot], wsems.at[slot])

        w_chunk_copy(0, 0).start()
        w_chunk_copy(1, 1).start()

        barrier = pltpu.get_barrier_semaphore()
        pl.semaphore_signal(barrier, inc=1, device_id=(left,),
                            device_id_type=pl.DeviceIdType.MESH)
        pl.semaphore_signal(barrier, inc=1, device_id=(right,),
                            device_id_type=pl.DeviceIdType.MESH)
        pl.semaphore_wait(barrier, 2)

        rows = lambda c: pl.ds(c * cr, cr)
        send_l = []
        send_r = []
        for c in range(NC):
            send_l.append(pltpu.make_async_remote_copy(
                src_ref=x_ref.at[rows(c)], dst_ref=buf_r.at[rows(c)],
                send_sem=ssl.at[c], recv_sem=rr.at[c],
                device_id=(left,), device_id_type=pl.DeviceIdType.MESH))
            send_r.append(pltpu.make_async_remote_copy(
                src_ref=x_ref.at[rows(c)], dst_ref=buf_l.at[rows(c)],
                send_sem=ssr.at[c], recv_sem=rl.at[c],
                device_id=(right,), device_id_type=pl.DeviceIdType.MESH))
            send_l[-1].start()
            send_r[-1].start()

        for c in range(n_wchunks):
            slot = c % 2
            w_chunk_copy(c, slot).wait()
            wq[:, c * ccols:(c + 1) * ccols] = wstage[slot].astype(FP8)
            if c + 2 < n_wchunks:
                w_chunk_copy(c + 2, slot).start()

        s = sx_ref[0] * sw_ref[0]

        def gemm_silu(ostage_slice, xin):
            acc = lax.dot_general(
                xin, wq[...],
                dimension_numbers=(((1,), (0,)), ((), ())),
                preferred_element_type=jnp.float32)
            y = acc * s
            ostage_slice[...] = y * jax.nn.sigmoid(y)

        def out_copy(slot, row0):
            return pltpu.make_async_copy(
                ostage.at[slot], out_ref.at[pl.ds(row0, m_per), :],
                osems.at[slot])

        gemm_silu(ostage.at[0], x_ref[...])
        cp_own = out_copy(0, my * m_per)
        cp_own.start()
        cp_own.wait()

        fwd_r = [pltpu.make_async_remote_copy(
            src_ref=buf_l.at[rows(c)], dst_ref=buf_o.at[rows(c)],
            send_sem=sfr.at[c], recv_sem=ro.at[c],
            device_id=(right,), device_id_type=pl.DeviceIdType.MESH)
            for c in range(2)]
        fwd_l = [pltpu.make_async_remote_copy(
            src_ref=buf_r.at[rows(c)], dst_ref=buf_o.at[rows(c)],
            send_sem=sfl.at[c - 2], recv_sem=ro.at[c],
            device_id=(left,), device_id_type=pl.DeviceIdType.MESH)
            for c in range(2, NC)]

        for c in range(NC):
            send_r[c].wait_recv()
            if c < 2:
                fwd_r[c].start()
            gemm_silu(ostage.at[1, rows(c)], buf_l[rows(c), :])
            send_l[c].wait_recv()
            if c >= 2:
                fwd_l[c - 2].start()
            gemm_silu(ostage.at[0, rows(c)], buf_r[rows(c), :])

        cp_left = out_copy(1, left * m_per)
        cp_left.start()
        cp_right = out_copy(0, right * m_per)
        cp_right.start()
        cp_left.wait()

        for c in (0, 2, 1, 3):
            fwd = fwd_r[c] if c < 2 else fwd_l[c - 2]
            fwd.wait_recv()
            gemm_silu(ostage.at[1, rows(c)], buf_o[rows(c), :])
        cp_opp = out_copy(1, opp * m_per)
        cp_opp.start()

        for d in send_l + send_r + fwd_r + fwd_l:
            d.wait_send()
        cp_right.wait()
        cp_opp.wait()

        @functools.partial(pl.run_scoped,
                           exit_sem=pltpu.SemaphoreType.REGULAR)
        def _(exit_sem):
            pl.semaphore_signal(exit_sem, inc=1, device_id=(left,),
                                device_id_type=pl.DeviceIdType.MESH)
            pl.semaphore_signal(exit_sem, inc=1, device_id=(right,),
                                device_id_type=pl.DeviceIdType.MESH)
            pl.semaphore_wait(exit_sem, 2)

    return pl.pallas_call(
        body,
        out_shape=jax.ShapeDtypeStruct((m_total, n_per), jnp.float32),
        in_specs=[
            pl.BlockSpec(memory_space=pltpu.VMEM),
            pl.BlockSpec(memory_space=pl.ANY),
            pl.BlockSpec(memory_space=pltpu.SMEM),
            pl.BlockSpec(memory_space=pltpu.SMEM),
            pl.BlockSpec(memory_space=pl.ANY),
        ],
        out_specs=pl.BlockSpec(memory_space=pl.ANY),
        input_output_aliases={4: 0},
        scratch_shapes=[
            pltpu.VMEM((2, k, ccols), jnp.float32),
            pltpu.VMEM((k, n_per), FP8),
            pltpu.VMEM((m_per, k), FP8),
            pltpu.VMEM((m_per, k), FP8),
            pltpu.VMEM((m_per, k), FP8),
            pltpu.VMEM((2, m_per, n_per), jnp.float32),
            pltpu.SemaphoreType.DMA((NC,)),
            pltpu.SemaphoreType.DMA((NC,)),
            pltpu.SemaphoreType.DMA((2,)),
            pltpu.SemaphoreType.DMA((2,)),
            pltpu.SemaphoreType.DMA((NC,)),
            pltpu.SemaphoreType.DMA((NC,)),
            pltpu.SemaphoreType.DMA((NC,)),
            pltpu.SemaphoreType.DMA((2,)),
            pltpu.SemaphoreType.DMA((2,)),
        ],
        compiler_params=pltpu.CompilerParams(
            collective_id=0, vmem_limit_bytes=64 * 1024 * 1024),
    )(xq, w_mat, scale_x, scale_w, out_init)


# device time: 107324 ns/iter; 1.1797x vs baseline; 1.1797x over previous
import functools

import jax
import jax.numpy as jnp
from jax import lax
from jax.experimental import pallas as pl
from jax.experimental.pallas import tpu as pltpu

N_DEV = 4
FP8 = jnp.float8_e4m3fn
NC = 4


def kernel(x, w_mat, scale_x, scale_w):
    m_per, k = x.shape
    n = w_mat.shape[1]
    n_per = n // N_DEV
    m_total = m_per * N_DEV
    cr = m_per // NC
    ccols = 256
    n_wchunks = n_per // ccols

    def body(x_ref, w_ref, out_ref,
             xstage, xq, wstage, wq, buf_l, buf_r, buf_o, ostage,
             ssl, ssr, sfr, sfl, rl, rr, ro, xsems, wsems, osems):
        my = lax.axis_index("i")
        left = lax.rem(my + N_DEV - 1, N_DEV)
        right = lax.rem(my + 1, N_DEV)
        opp = lax.rem(my + 2, N_DEV)

        rows = lambda c: pl.ds(c * cr, cr)

        def x_chunk_copy(c, slot):
            return pltpu.make_async_copy(
                x_ref.at[rows(c)], xstage.at[slot], xsems.at[slot])

        def w_chunk_copy(c, slot):
            return pltpu.make_async_copy(
                w_ref.at[:, pl.ds(my * n_per + c * ccols, ccols)],
                wstage.at[slot], wsems.at[slot])

        x_chunk_copy(0, 0).start()
        x_chunk_copy(1, 1).start()
        w_chunk_copy(0, 0).start()
        w_chunk_copy(1, 1).start()

        barrier = pltpu.get_barrier_semaphore()
        pl.semaphore_signal(barrier, inc=1, device_id=(left,),
                            device_id_type=pl.DeviceIdType.MESH)
        pl.semaphore_signal(barrier, inc=1, device_id=(right,),
                            device_id_type=pl.DeviceIdType.MESH)
        pl.semaphore_wait(barrier, 2)

        send_l = []
        send_r = []
        for c in range(NC):
            slot = c % 2
            x_chunk_copy(c, slot).wait()
            xq[rows(c), :] = xstage[slot].astype(FP8)
            if c + 2 < NC:
                x_chunk_copy(c + 2, slot).start()
            send_l.append(pltpu.make_async_remote_copy(
                src_ref=xq.at[rows(c)], dst_ref=buf_r.at[rows(c)],
                send_sem=ssl.at[c], recv_sem=rr.at[c],
                device_id=(left,), device_id_type=pl.DeviceIdType.MESH))
            send_r.append(pltpu.make_async_remote_copy(
                src_ref=xq.at[rows(c)], dst_ref=buf_l.at[rows(c)],
                send_sem=ssr.at[c], recv_sem=rl.at[c],
                device_id=(right,), device_id_type=pl.DeviceIdType.MESH))
            send_l[-1].start()
            send_r[-1].start()

        for c in range(n_wchunks):
            slot = c % 2
            w_chunk_copy(c, slot).wait()
            wq[:, c * ccols:(c + 1) * ccols] = wstage[slot].astype(FP8)
            if c + 2 < n_wchunks:
                w_chunk_copy(c + 2, slot).start()

        def gemm(ostage_slice, xin):
            ostage_slice[...] = lax.dot_general(
                xin, wq[...],
                dimension_numbers=(((1,), (0,)), ((), ())),
                preferred_element_type=jnp.float32)

        def out_copy(slot, row0):
            return pltpu.make_async_copy(
                ostage.at[slot], out_ref.at[pl.ds(row0, m_per), :],
                osems.at[slot])

        gemm(ostage.at[0], xq[...])
        cp_own = out_copy(0, my * m_per)
        cp_own.start()
        cp_own.wait()

        fwd_r = [pltpu.make_async_remote_copy(
            src_ref=buf_l.at[rows(c)], dst_ref=buf_o.at[rows(c)],
            send_sem=sfr.at[c], recv_sem=ro.at[c],
            device_id=(right,), device_id_type=pl.DeviceIdType.MESH)
            for c in range(2)]
        fwd_l = [pltpu.make_async_remote_copy(
            src_ref=buf_r.at[rows(c)], dst_ref=buf_o.at[rows(c)],
            send_sem=sfl.at[c - 2], recv_sem=ro.at[c],
            device_id=(left,), device_id_type=pl.DeviceIdType.MESH)
            for c in range(2, NC)]

        for c in range(NC):
            send_r[c].wait_recv()
            if c < 2:
                fwd_r[c].start()
            gemm(ostage.at[1, rows(c)], buf_l[rows(c), :])
            send_l[c].wait_recv()
            if c >= 2:
                fwd_l[c - 2].start()
            gemm(ostage.at[0, rows(c)], buf_r[rows(c), :])

        cp_left = out_copy(1, left * m_per)
        cp_left.start()
        cp_right = out_copy(0, right * m_per)
        cp_right.start()
        cp_left.wait()

        for c in (0, 2, 1, 3):
            fwd = fwd_r[c] if c < 2 else fwd_l[c - 2]
            fwd.wait_recv()
            gemm(ostage.at[1, rows(c)], buf_o[rows(c), :])
        cp_opp = out_copy(1, opp * m_per)
        cp_opp.start()

        for d in send_l + send_r + fwd_r + fwd_l:
            d.wait_send()
        cp_right.wait()
        cp_opp.wait()

        @functools.partial(pl.run_scoped,
                           exit_sem=pltpu.SemaphoreType.REGULAR)
        def _(exit_sem):
            pl.semaphore_signal(exit_sem, inc=1, device_id=(left,),
                                device_id_type=pl.DeviceIdType.MESH)
            pl.semaphore_signal(exit_sem, inc=1, device_id=(right,),
                                device_id_type=pl.DeviceIdType.MESH)
            pl.semaphore_wait(exit_sem, 2)

    acc = pl.pallas_call(
        body,
        out_shape=jax.ShapeDtypeStruct((m_total, n_per), jnp.float32),
        in_specs=[
            pl.BlockSpec(memory_space=pl.ANY),
            pl.BlockSpec(memory_space=pl.ANY),
        ],
        out_specs=pl.BlockSpec(memory_space=pl.ANY),
        scratch_shapes=[
            pltpu.VMEM((2, cr, k), jnp.float32),
            pltpu.VMEM((m_per, k), FP8),
            pltpu.VMEM((2, k, ccols), jnp.float32),
            pltpu.VMEM((k, n_per), FP8),
            pltpu.VMEM((m_per, k), FP8),
            pltpu.VMEM((m_per, k), FP8),
            pltpu.VMEM((m_per, k), FP8),
            pltpu.VMEM((2, m_per, n_per), jnp.float32),
            pltpu.SemaphoreType.DMA((NC,)),
            pltpu.SemaphoreType.DMA((NC,)),
            pltpu.SemaphoreType.DMA((2,)),
            pltpu.SemaphoreType.DMA((2,)),
            pltpu.SemaphoreType.DMA((NC,)),
            pltpu.SemaphoreType.DMA((NC,)),
            pltpu.SemaphoreType.DMA((NC,)),
            pltpu.SemaphoreType.DMA((2,)),
            pltpu.SemaphoreType.DMA((2,)),
            pltpu.SemaphoreType.DMA((2,)),
        ],
        compiler_params=pltpu.CompilerParams(
            collective_id=0, vmem_limit_bytes=64 * 1024 * 1024),
    )(x, w_mat)

    s = (scale_x * scale_w).astype(jnp.float32)
    rows_blk = 512

    def silu_body(s_ref, acc_ref, o_ref):
        y = acc_ref[...] * s_ref[0]
        o_ref[...] = y * jax.nn.sigmoid(y)

    return pl.pallas_call(
        silu_body,
        grid=(m_total // rows_blk,),
        in_specs=[
            pl.BlockSpec(memory_space=pltpu.SMEM),
            pl.BlockSpec((rows_blk, n_per), lambda i: (i, 0)),
        ],
        out_specs=pl.BlockSpec((rows_blk, n_per), lambda i: (i, 0)),
        out_shape=jax.ShapeDtypeStruct((m_total, n_per), jnp.float32),
        compiler_params=pltpu.CompilerParams(
            vmem_limit_bytes=64 * 1024 * 1024),
    )(s, acc)


# device time: 101740 ns/iter; 1.2445x vs baseline; 1.0549x over previous
import functools

import jax
import jax.numpy as jnp
from jax import lax
from jax.experimental import pallas as pl
from jax.experimental.pallas import tpu as pltpu

N_DEV = 4
FP8 = jnp.float8_e4m3fn
NC = 4


def kernel(x, w_mat, scale_x, scale_w):
    m_per, k = x.shape
    n = w_mat.shape[1]
    n_per = n // N_DEV
    m_total = m_per * N_DEV
    cr = m_per // NC
    ccols = 256
    n_wchunks = n_per // ccols

    def body(x_ref, w_ref, out_ref,
             xstage, xq, wstage, wq, buf_l, buf_r, buf_o, ostage,
             ssl, ssr, sfr, sfl, rl, rr, ro, xsems, wsems, osems):
        my = lax.axis_index("i")
        left = lax.rem(my + N_DEV - 1, N_DEV)
        right = lax.rem(my + 1, N_DEV)
        opp = lax.rem(my + 2, N_DEV)

        rows = lambda c: pl.ds(c * cr, cr)

        def x_chunk_copy(c, slot):
            return pltpu.make_async_copy(
                x_ref.at[rows(c)], xstage.at[slot], xsems.at[slot])

        def w_chunk_copy(c, slot):
            return pltpu.make_async_copy(
                w_ref.at[:, pl.ds(my * n_per + c * ccols, ccols)],
                wstage.at[slot], wsems.at[slot])

        x_chunk_copy(0, 0).start()
        x_chunk_copy(1, 1).start()
        for c in range(4):
            w_chunk_copy(c, c).start()

        barrier = pltpu.get_barrier_semaphore()
        pl.semaphore_signal(barrier, inc=1, device_id=(left,),
                            device_id_type=pl.DeviceIdType.MESH)
        pl.semaphore_signal(barrier, inc=1, device_id=(right,),
                            device_id_type=pl.DeviceIdType.MESH)
        pl.semaphore_wait(barrier, 2)

        send_l = []
        send_r = []
        for c in range(NC):
            slot = c % 2
            x_chunk_copy(c, slot).wait()
            xq[rows(c), :] = xstage[slot].astype(FP8)
            if c + 2 < NC:
                x_chunk_copy(c + 2, slot).start()
            send_l.append(pltpu.make_async_remote_copy(
                src_ref=xq.at[rows(c)], dst_ref=buf_r.at[rows(c)],
                send_sem=ssl.at[c], recv_sem=rr.at[c],
                device_id=(left,), device_id_type=pl.DeviceIdType.MESH))
            send_r.append(pltpu.make_async_remote_copy(
                src_ref=xq.at[rows(c)], dst_ref=buf_l.at[rows(c)],
                send_sem=ssr.at[c], recv_sem=rl.at[c],
                device_id=(right,), device_id_type=pl.DeviceIdType.MESH))
            send_l[-1].start()
            send_r[-1].start()

        for c in range(n_wchunks):
            slot = c % 4
            w_chunk_copy(c, slot).wait()
            wq[:, c * ccols:(c + 1) * ccols] = wstage[slot].astype(FP8)
            if c + 4 < n_wchunks:
                w_chunk_copy(c + 4, slot).start()

        def gemm(ostage_slice, xin):
            ostage_slice[...] = lax.dot_general(
                xin, wq[...],
                dimension_numbers=(((1,), (0,)), ((), ())),
                preferred_element_type=jnp.float32).astype(jnp.bfloat16)

        def out_copy(slot, row0):
            return pltpu.make_async_copy(
                ostage.at[slot], out_ref.at[pl.ds(row0, m_per), :],
                osems.at[slot])

        gemm(ostage.at[0], xq[...])
        cp_own = out_copy(0, my * m_per)
        cp_own.start()

        fwd_r = [pltpu.make_async_remote_copy(
            src_ref=buf_l.at[rows(c)], dst_ref=buf_o.at[rows(c)],
            send_sem=sfr.at[c], recv_sem=ro.at[c],
            device_id=(right,), device_id_type=pl.DeviceIdType.MESH)
            for c in range(2)]
        fwd_l = [pltpu.make_async_remote_copy(
            src_ref=buf_r.at[rows(c)], dst_ref=buf_o.at[rows(c)],
            send_sem=sfl.at[c - 2], recv_sem=ro.at[c],
            device_id=(left,), device_id_type=pl.DeviceIdType.MESH)
            for c in range(2, NC)]

        for c in range(NC):
            send_r[c].wait_recv()
            if c < 2:
                fwd_r[c].start()
            gemm(ostage.at[1, rows(c)], buf_l[rows(c), :])
            send_l[c].wait_recv()
            if c >= 2:
                fwd_l[c - 2].start()
            if c == 0:
                cp_own.wait()
            gemm(ostage.at[0, rows(c)], buf_r[rows(c), :])

        cp_left = out_copy(1, left * m_per)
        cp_left.start()
        cp_right = out_copy(0, right * m_per)
        cp_right.start()
        cp_left.wait()

        for c in (0, 2, 1, 3):
            fwd = fwd_r[c] if c < 2 else fwd_l[c - 2]
            fwd.wait_recv()
            gemm(ostage.at[1, rows(c)], buf_o[rows(c), :])
        cp_opp = out_copy(1, opp * m_per)
        cp_opp.start()

        for d in send_l + send_r + fwd_r + fwd_l:
            d.wait_send()
        cp_right.wait()
        cp_opp.wait()

        @functools.partial(pl.run_scoped,
                           exit_sem=pltpu.SemaphoreType.REGULAR)
        def _(exit_sem):
            pl.semaphore_signal(exit_sem, inc=1, device_id=(left,),
                                device_id_type=pl.DeviceIdType.MESH)
            pl.semaphore_signal(exit_sem, inc=1, device_id=(right,),
                                device_id_type=pl.DeviceIdType.MESH)
            pl.semaphore_wait(exit_sem, 2)

    acc = pl.pallas_call(
        body,
        out_shape=jax.ShapeDtypeStruct((m_total, n_per), jnp.bfloat16),
        in_specs=[
            pl.BlockSpec(memory_space=pl.ANY),
            pl.BlockSpec(memory_space=pl.ANY),
        ],
        out_specs=pl.BlockSpec(memory_space=pl.ANY),
        scratch_shapes=[
            pltpu.VMEM((2, cr, k), jnp.float32),
            pltpu.VMEM((m_per, k), FP8),
            pltpu.VMEM((4, k, ccols), jnp.float32),
            pltpu.VMEM((k, n_per), FP8),
            pltpu.VMEM((m_per, k), FP8),
            pltpu.VMEM((m_per, k), FP8),
            pltpu.VMEM((m_per, k), FP8),
            pltpu.VMEM((2, m_per, n_per), jnp.bfloat16),
            pltpu.SemaphoreType.DMA((NC,)),
            pltpu.SemaphoreType.DMA((NC,)),
            pltpu.SemaphoreType.DMA((2,)),
            pltpu.SemaphoreType.DMA((2,)),
            pltpu.SemaphoreType.DMA((NC,)),
            pltpu.SemaphoreType.DMA((NC,)),
            pltpu.SemaphoreType.DMA((NC,)),
            pltpu.SemaphoreType.DMA((2,)),
            pltpu.SemaphoreType.DMA((4,)),
            pltpu.SemaphoreType.DMA((2,)),
        ],
        compiler_params=pltpu.CompilerParams(
            collective_id=0, vmem_limit_bytes=64 * 1024 * 1024),
    )(x, w_mat)

    s = (scale_x * scale_w).astype(jnp.float32)
    rows_blk = 512

    def silu_body(s_ref, acc_ref, o_ref):
        y = acc_ref[...].astype(jnp.float32) * s_ref[0]
        o_ref[...] = y * jax.nn.sigmoid(y)

    return pl.pallas_call(
        silu_body,
        grid=(m_total // rows_blk,),
        in_specs=[
            pl.BlockSpec(memory_space=pltpu.SMEM),
            pl.BlockSpec((rows_blk, n_per), lambda i: (i, 0)),
        ],
        out_specs=pl.BlockSpec((rows_blk, n_per), lambda i: (i, 0)),
        out_shape=jax.ShapeDtypeStruct((m_total, n_per), jnp.float32),
        compiler_params=pltpu.CompilerParams(
            vmem_limit_bytes=64 * 1024 * 1024),
    )(s, acc)


# device time: 98874 ns/iter; 1.2805x vs baseline; 1.0290x over previous
import functools

import jax
import jax.numpy as jnp
from jax import lax
from jax.experimental import pallas as pl
from jax.experimental.pallas import tpu as pltpu

N_DEV = 4
FP8 = jnp.float8_e4m3fn
NC = 8
NH = NC // 2


def kernel(x, w_mat, scale_x, scale_w):
    m_per, k = x.shape
    n = w_mat.shape[1]
    n_per = n // N_DEV
    m_total = m_per * N_DEV
    cr = m_per // NC
    ccols = 256
    n_wchunks = n_per // ccols

    def body(x_ref, w_ref, out_ref,
             xstage, xq, wstage, wq, buf_l, buf_r, buf_o, ostage,
             ssl, ssr, sfr, sfl, rl, rr, ro, xsems, wsems, osems):
        my = lax.axis_index("i")
        left = lax.rem(my + N_DEV - 1, N_DEV)
        right = lax.rem(my + 1, N_DEV)
        opp = lax.rem(my + 2, N_DEV)

        rows = lambda c: pl.ds(c * cr, cr)

        def x_chunk_copy(c, slot):
            return pltpu.make_async_copy(
                x_ref.at[rows(c)], xstage.at[slot], xsems.at[slot])

        def w_chunk_copy(c, slot):
            return pltpu.make_async_copy(
                w_ref.at[:, pl.ds(my * n_per + c * ccols, ccols)],
                wstage.at[slot], wsems.at[slot])

        x_chunk_copy(0, 0).start()
        x_chunk_copy(1, 1).start()
        for c in range(4):
            w_chunk_copy(c, c).start()

        barrier = pltpu.get_barrier_semaphore()
        pl.semaphore_signal(barrier, inc=1, device_id=(left,),
                            device_id_type=pl.DeviceIdType.MESH)
        pl.semaphore_signal(barrier, inc=1, device_id=(right,),
                            device_id_type=pl.DeviceIdType.MESH)
        pl.semaphore_wait(barrier, 2)

        send_l = []
        send_r = []
        for c in range(NC):
            slot = c % 2
            x_chunk_copy(c, slot).wait()
            xq[rows(c), :] = xstage[slot].astype(FP8)
            if c + 2 < NC:
                x_chunk_copy(c + 2, slot).start()
            send_l.append(pltpu.make_async_remote_copy(
                src_ref=xq.at[rows(c)], dst_ref=buf_r.at[rows(c)],
                send_sem=ssl.at[c], recv_sem=rr.at[c],
                device_id=(left,), device_id_type=pl.DeviceIdType.MESH))
            send_r.append(pltpu.make_async_remote_copy(
                src_ref=xq.at[rows(c)], dst_ref=buf_l.at[rows(c)],
                send_sem=ssr.at[c], recv_sem=rl.at[c],
                device_id=(right,), device_id_type=pl.DeviceIdType.MESH))
            send_l[-1].start()
            send_r[-1].start()

        for c in range(n_wchunks):
            slot = c % 4
            w_chunk_copy(c, slot).wait()
            wq[:, c * ccols:(c + 1) * ccols] = wstage[slot].astype(FP8)
            if c + 4 < n_wchunks:
                w_chunk_copy(c + 4, slot).start()

        def gemm(ostage_slice, xin):
            ostage_slice[...] = lax.dot_general(
                xin, wq[...],
                dimension_numbers=(((1,), (0,)), ((), ())),
                preferred_element_type=jnp.float32).astype(jnp.bfloat16)

        def out_copy(slot, row0):
            return pltpu.make_async_copy(
                ostage.at[slot], out_ref.at[pl.ds(row0, m_per), :],
                osems.at[slot])

        gemm(ostage.at[0], xq[...])
        cp_own = out_copy(0, my * m_per)
        cp_own.start()

        fwd_r = [pltpu.make_async_remote_copy(
            src_ref=buf_l.at[rows(c)], dst_ref=buf_o.at[rows(c)],
            send_sem=sfr.at[c], recv_sem=ro.at[c],
            device_id=(right,), device_id_type=pl.DeviceIdType.MESH)
            for c in range(NH)]
        fwd_l = [pltpu.make_async_remote_copy(
            src_ref=buf_r.at[rows(c)], dst_ref=buf_o.at[rows(c)],
            send_sem=sfl.at[c - NH], recv_sem=ro.at[c],
            device_id=(left,), device_id_type=pl.DeviceIdType.MESH)
            for c in range(NH, NC)]

        for c in range(NC):
            send_r[c].wait_recv()
            if c < NH:
                fwd_r[c].start()
            gemm(ostage.at[1, rows(c)], buf_l[rows(c), :])
            send_l[c].wait_recv()
            if c >= NH:
                fwd_l[c - NH].start()
            if c == 0:
                cp_own.wait()
            gemm(ostage.at[0, rows(c)], buf_r[rows(c), :])

        cp_left = out_copy(1, left * m_per)
        cp_left.start()
        cp_right = out_copy(0, right * m_per)
        cp_right.start()
        cp_left.wait()

        opp_order = [c for pair in zip(range(NH), range(NH, NC))
                     for c in pair]
        for c in opp_order:
            fwd = fwd_r[c] if c < NH else fwd_l[c - NH]
            fwd.wait_recv()
            gemm(ostage.at[1, rows(c)], buf_o[rows(c), :])
        cp_opp = out_copy(1, opp * m_per)
        cp_opp.start()

        for d in send_l + send_r + fwd_r + fwd_l:
            d.wait_send()
        cp_right.wait()
        cp_opp.wait()

        @functools.partial(pl.run_scoped,
                           exit_sem=pltpu.SemaphoreType.REGULAR)
        def _(exit_sem):
            pl.semaphore_signal(exit_sem, inc=1, device_id=(left,),
                                device_id_type=pl.DeviceIdType.MESH)
            pl.semaphore_signal(exit_sem, inc=1, device_id=(right,),
                                device_id_type=pl.DeviceIdType.MESH)
            pl.semaphore_wait(exit_sem, 2)

    acc = pl.pallas_call(
        body,
        out_shape=jax.ShapeDtypeStruct((m_total, n_per), jnp.bfloat16),
        in_specs=[
            pl.BlockSpec(memory_space=pl.ANY),
            pl.BlockSpec(memory_space=pl.ANY),
        ],
        out_specs=pl.BlockSpec(memory_space=pl.ANY),
        scratch_shapes=[
            pltpu.VMEM((2, cr, k), jnp.float32),
            pltpu.VMEM((m_per, k), FP8),
            pltpu.VMEM((4, k, ccols), jnp.float32),
            pltpu.VMEM((k, n_per), FP8),
            pltpu.VMEM((m_per, k), FP8),
            pltpu.VMEM((m_per, k), FP8),
            pltpu.VMEM((m_per, k), FP8),
            pltpu.VMEM((2, m_per, n_per), jnp.bfloat16),
            pltpu.SemaphoreType.DMA((NC,)),
            pltpu.SemaphoreType.DMA((NC,)),
            pltpu.SemaphoreType.DMA((NH,)),
            pltpu.SemaphoreType.DMA((NH,)),
            pltpu.SemaphoreType.DMA((NC,)),
            pltpu.SemaphoreType.DMA((NC,)),
            pltpu.SemaphoreType.DMA((NC,)),
            pltpu.SemaphoreType.DMA((2,)),
            pltpu.SemaphoreType.DMA((4,)),
            pltpu.SemaphoreType.DMA((2,)),
        ],
        compiler_params=pltpu.CompilerParams(
            collective_id=0, vmem_limit_bytes=64 * 1024 * 1024),
    )(x, w_mat)

    s = (scale_x * scale_w).astype(jnp.float32)
    rows_blk = 512

    def silu_body(s_ref, acc_ref, o_ref):
        y = acc_ref[...].astype(jnp.float32) * s_ref[0]
        o_ref[...] = y * jax.nn.sigmoid(y)

    return pl.pallas_call(
        silu_body,
        grid=(m_total // rows_blk,),
        in_specs=[
            pl.BlockSpec(memory_space=pltpu.SMEM),
            pl.BlockSpec((rows_blk, n_per), lambda i: (i, 0)),
        ],
        out_specs=pl.BlockSpec((rows_blk, n_per), lambda i: (i, 0)),
        out_shape=jax.ShapeDtypeStruct((m_total, n_per), jnp.float32),
        compiler_params=pltpu.CompilerParams(
            vmem_limit_bytes=64 * 1024 * 1024),
    )(s, acc)
